# baseline (speedup 1.0000x reference)
import functools

import jax
import jax.numpy as jnp
from jax import lax
from jax.experimental import pallas as pl
from jax.experimental.pallas import tpu as pltpu
from jax.experimental.pallas import tpu_sc as plsc

_ROWS, _N = 64, 8192
_L = 16
_NC, _NS = 2, 16
_NW = _NC * _NS
_RPW = _ROWS // _NW
_NV = _N // _L
_NH = _NV // 2
_BISECT_ITERS = 20
_NACC = 4
_G = 8


def _butterfly(v, op):
    iota = lax.iota(jnp.int32, _L)
    for k in (8, 4, 2, 1):
        v = op(v, v.at[iota ^ k].get(mode="promise_in_bounds"))
    return v


def _sc_body(x_hbm, out_hbm, buf, compa, compb):
    cid = lax.axis_index("c")
    sid = lax.axis_index("s")
    wid = sid * _NC + cid
    row0 = wid * _RPW
    pltpu.sync_copy(x_hbm.at[pl.ds(row0, _RPW)], buf)

    zero = jnp.zeros((_L,), jnp.float32)

    for r in range(_RPW):
        row = buf.at[r]

        def max_body(j, accs):
            base = j * (_NACC * _L)
            return tuple(
                jnp.maximum(accs[t], row[pl.ds(base + t * _L, _L)])
                for t in range(_NACC))

        accs = lax.fori_loop(
            1, _NV // _NACC, max_body,
            tuple(row[pl.ds(t * _L, _L)] for t in range(_NACC)), unroll=4)
        m = _butterfly(jnp.maximum(jnp.maximum(accs[0], accs[1]),
                                   jnp.maximum(accs[2], accs[3])),
                       jnp.maximum)
        thr = m - 1.0
        thr_s = thr[0]

        izero = jnp.zeros((_L,), jnp.int32)

        def comp_body(j, carry):
            offa, offb = carry
            base = j * (_G * _L)
            va = [row[pl.ds(base + t * _L, _L)] for t in range(_G)]
            vb = [row[pl.ds(_NH * _L + base + t * _L, _L)]
                  for t in range(_G)]
            code_a, code_b = izero, izero
            for t in range(_G):
                code_a = code_a | jnp.where(va[t] > thr, 1 << t, 0)
                code_b = code_b | jnp.where(vb[t] > thr, 1 << t, 0)
            ca = _butterfly(code_a, jnp.bitwise_or)[0]
            cb = _butterfly(code_b, jnp.bitwise_or)[0]
            for t in range(_G):
                compa[pl.ds(pl.multiple_of(offa, _L), _L)] = va[t]
                compb[pl.ds(pl.multiple_of(offb, _L), _L)] = vb[t]
                offa = offa + jnp.where(ca & (1 << t) != 0, _L, 0)
                offb = offb + jnp.where(cb & (1 << t) != 0, _L, 0)
            return offa, offb

        offa, offb = lax.fori_loop(0, _NH // _G, comp_body, (0, 0),
                                   unroll=2)

        sent = jnp.full((_L,), -3e38, jnp.float32)
        for t in range(_G):
            compa[pl.ds(pl.multiple_of(offa, _L) + t * _L, _L)] = sent
            compb[pl.ds(pl.multiple_of(offb, _L) + t * _L, _L)] = sent
        na_blocks = (offa + (_G * _L - 1)) // (_G * _L)
        nb_blocks = (offb + (_G * _L - 1)) // (_G * _L)

        def bis_body(i, carry):
            lo, hi = carry
            mid = 0.5 * (lo + hi)

            def inner_a(k, a):
                base = pl.multiple_of(k * (_G * _L), _L)
                for t in range(_G):
                    a = a + jnp.maximum(
                        compa[pl.ds(base + t * _L, _L)] - mid, 0.0)
                return a

            def inner_b(k, a):
                base = pl.multiple_of(k * (_G * _L), _L)
                for t in range(_G):
                    a = a + jnp.maximum(
                        compb[pl.ds(base + t * _L, _L)] - mid, 0.0)
                return a

            a = lax.fori_loop(0, na_blocks, inner_a, zero)
            a = lax.fori_loop(0, nb_blocks, inner_b, a)
            s = _butterfly(a, jnp.add)
            pred = s >= 1.0
            return jnp.where(pred, mid, lo), jnp.where(pred, hi, mid)

        lo, _hi = lax.fori_loop(0, _BISECT_ITERS, bis_body, (thr, m))

        def fin_a(k, carry):
            sa, ka = carry
            base = pl.multiple_of(k * (_G * _L), _L)
            for t in range(_G):
                v = compa[pl.ds(base + t * _L, _L)]
                sup = v > lo
                sa = sa + jnp.where(sup, v, 0.0)
                ka = ka + jnp.where(sup, 1.0, 0.0)
            return sa, ka

        def fin_b(k, carry):
            sa, ka = carry
            base = pl.multiple_of(k * (_G * _L), _L)
            for t in range(_G):
                v = compb[pl.ds(base + t * _L, _L)]
                sup = v > lo
                sa = sa + jnp.where(sup, v, 0.0)
                ka = ka + jnp.where(sup, 1.0, 0.0)
            return sa, ka

        sa, ka = lax.fori_loop(0, na_blocks, fin_a, (zero, zero))
        sa, ka = lax.fori_loop(0, nb_blocks, fin_b, (sa, ka))
        tau = (_butterfly(sa, jnp.add) - 1.0) / _butterfly(ka, jnp.add)

        def out_body(j, carry):
            base = j * (_NACC * _L)
            for t in range(_NACC):
                sl = pl.ds(base + t * _L, _L)
                row[sl] = jnp.maximum(row[sl] - tau, 0.0)
            return carry

        lax.fori_loop(0, _NV // _NACC, out_body, 0, unroll=4)

    pltpu.sync_copy(buf, out_hbm.at[pl.ds(row0, _RPW)])


@functools.partial(
    pl.kernel,
    out_type=jax.ShapeDtypeStruct((_ROWS, _N), jnp.float32),
    mesh=plsc.VectorSubcoreMesh(core_axis_name="c", subcore_axis_name="s",
                                num_cores=_NC, num_subcores=_NS),
    scratch_types=[pltpu.VMEM((_RPW, _N), jnp.float32),
                   pltpu.VMEM((_N + 2 * _G * _L, ), jnp.float32),
                   pltpu.VMEM((_NH * _L + _G * _L,), jnp.float32)],
)
def _sparsemax_sc(x_hbm, out_hbm, buf, compa, compb):
    _sc_body(x_hbm, out_hbm, buf, compa, compb)


@jax.jit
def kernel(input):
    return _sparsemax_sc(input)


# speedup vs baseline: 1.0488x; 1.0488x over previous
import functools

import jax
import jax.numpy as jnp
from jax import lax
from jax.experimental import pallas as pl
from jax.experimental.pallas import tpu as pltpu
from jax.experimental.pallas import tpu_sc as plsc

_ROWS, _N = 64, 8192
_L = 16
_NC, _NS = 2, 16
_NW = _NC * _NS
_RPW = _ROWS // _NW
_NV = _N // _L
_NH = _NV // 2
_BISECT_ITERS = 16
_NACC = 4
_G = 8


def _butterfly(v, op):
    iota = lax.iota(jnp.int32, _L)
    for k in (8, 4, 2, 1):
        v = op(v, v.at[iota ^ k].get(mode="promise_in_bounds"))
    return v


def _sc_body(x_hbm, out_hbm, buf, compa, compb):
    cid = lax.axis_index("c")
    sid = lax.axis_index("s")
    wid = sid * _NC + cid
    row0 = wid * _RPW
    pltpu.sync_copy(x_hbm.at[pl.ds(row0, _RPW)], buf)

    zero = jnp.zeros((_L,), jnp.float32)

    for r in range(_RPW):
        row = buf.at[r]

        def max_body(j, accs):
            base = j * (_NACC * _L)
            return tuple(
                jnp.maximum(accs[t], row[pl.ds(base + t * _L, _L)])
                for t in range(_NACC))

        accs = lax.fori_loop(
            1, _NV // _NACC, max_body,
            tuple(row[pl.ds(t * _L, _L)] for t in range(_NACC)), unroll=4)
        m = _butterfly(jnp.maximum(jnp.maximum(accs[0], accs[1]),
                                   jnp.maximum(accs[2], accs[3])),
                       jnp.maximum)
        thr = m - 1.0
        thr_s = thr[0]

        izero = jnp.zeros((_L,), jnp.int32)

        _CG = 16

        def comp_body(j, carry):
            offa, offb = carry
            base = j * (_CG * _L)
            va = [row[pl.ds(base + t * _L, _L)] for t in range(_CG)]
            vb = [row[pl.ds(_NH * _L + base + t * _L, _L)]
                  for t in range(_CG)]
            code_a, code_b = izero, izero
            for t in range(_CG):
                code_a = code_a | jnp.where(va[t] > thr, 1 << t, 0)
                code_b = code_b | jnp.where(vb[t] > thr, 1 << t, 0)
            ca = _butterfly(code_a, jnp.bitwise_or)[0] << 4
            cb = _butterfly(code_b, jnp.bitwise_or)[0] << 4
            for t in range(_CG):
                compa[pl.ds(pl.multiple_of(offa, _L), _L)] = va[t]
                compb[pl.ds(pl.multiple_of(offb, _L), _L)] = vb[t]
                offa = offa + ((ca >> t) & _L)
                offb = offb + ((cb >> t) & _L)
            return offa, offb

        offa, offb = lax.fori_loop(0, _NH // _CG, comp_body, (0, 0),
                                   unroll=2)

        sent = jnp.full((_L,), -3e38, jnp.float32)
        for t in range(_G):
            compa[pl.ds(pl.multiple_of(offa, _L) + t * _L, _L)] = sent
            compb[pl.ds(pl.multiple_of(offb, _L) + t * _L, _L)] = sent
        na_blocks = (offa + (_G * _L - 1)) // (_G * _L)
        nb_blocks = (offb + (_G * _L - 1)) // (_G * _L)

        def bis_body(i, carry):
            lo, hi = carry
            mid = 0.5 * (lo + hi)

            def inner_a(k, a):
                base = pl.multiple_of(k * (_G * _L), _L)
                for t in range(_G):
                    a = a + jnp.maximum(
                        compa[pl.ds(base + t * _L, _L)] - mid, 0.0)
                return a

            def inner_b(k, a):
                base = pl.multiple_of(k * (_G * _L), _L)
                for t in range(_G):
                    a = a + jnp.maximum(
                        compb[pl.ds(base + t * _L, _L)] - mid, 0.0)
                return a

            a = lax.fori_loop(0, na_blocks, inner_a, zero)
            a = lax.fori_loop(0, nb_blocks, inner_b, a)
            s = _butterfly(a, jnp.add)
            pred = s >= 1.0
            return jnp.where(pred, mid, lo), jnp.where(pred, hi, mid)

        lo, _hi = lax.fori_loop(0, _BISECT_ITERS, bis_body, (thr, m))

        def fin_a(k, carry):
            sa, ka = carry
            base = pl.multiple_of(k * (_G * _L), _L)
            for t in range(_G):
                v = compa[pl.ds(base + t * _L, _L)]
                sup = v > lo
                sa = sa + jnp.where(sup, v, 0.0)
                ka = ka + jnp.where(sup, 1.0, 0.0)
            return sa, ka

        def fin_b(k, carry):
            sa, ka = carry
            base = pl.multiple_of(k * (_G * _L), _L)
            for t in range(_G):
                v = compb[pl.ds(base + t * _L, _L)]
                sup = v > lo
                sa = sa + jnp.where(sup, v, 0.0)
                ka = ka + jnp.where(sup, 1.0, 0.0)
            return sa, ka

        sa, ka = lax.fori_loop(0, na_blocks, fin_a, (zero, zero))
        sa, ka = lax.fori_loop(0, nb_blocks, fin_b, (sa, ka))
        tau = (_butterfly(sa, jnp.add) - 1.0) / _butterfly(ka, jnp.add)

        def out_body(j, carry):
            base = j * (_NACC * _L)
            for t in range(_NACC):
                sl = pl.ds(base + t * _L, _L)
                row[sl] = jnp.maximum(row[sl] - tau, 0.0)
            return carry

        lax.fori_loop(0, _NV // _NACC, out_body, 0, unroll=4)

    pltpu.sync_copy(buf, out_hbm.at[pl.ds(row0, _RPW)])


@functools.partial(
    pl.kernel,
    out_type=jax.ShapeDtypeStruct((_ROWS, _N), jnp.float32),
    mesh=plsc.VectorSubcoreMesh(core_axis_name="c", subcore_axis_name="s",
                                num_cores=_NC, num_subcores=_NS),
    scratch_types=[pltpu.VMEM((_RPW, _N), jnp.float32),
                   pltpu.VMEM((_N + 2 * _G * _L, ), jnp.float32),
                   pltpu.VMEM((_NH * _L + _G * _L,), jnp.float32)],
)
def _sparsemax_sc(x_hbm, out_hbm, buf, compa, compb):
    _sc_body(x_hbm, out_hbm, buf, compa, compb)


@jax.jit
def kernel(input):
    return _sparsemax_sc(input)


# speedup vs baseline: 1.0548x; 1.0057x over previous
import functools

import jax
import jax.numpy as jnp
from jax import lax
from jax.experimental import pallas as pl
from jax.experimental.pallas import tpu as pltpu
from jax.experimental.pallas import tpu_sc as plsc

_ROWS, _N = 64, 8192
_L = 16
_NC, _NS = 2, 16
_NW = _NC * _NS
_RPW = _ROWS // _NW
_NV = _N // _L
_NH = _NV // 2
_BISECT_ITERS = 16
_NACC = 4
_G = 8


def _butterfly(v, op):
    iota = lax.iota(jnp.int32, _L)
    for k in (8, 4, 2, 1):
        v = op(v, v.at[iota ^ k].get(mode="promise_in_bounds"))
    return v


def _sc_body(x_hbm, out_hbm, buf, compa, compb, sems):
    cid = lax.axis_index("c")
    sid = lax.axis_index("s")
    wid = sid * _NC + cid
    row0 = wid * _RPW

    in_copies = [
        pltpu.make_async_copy(x_hbm.at[pl.ds(row0 + r, 1)],
                              buf.at[pl.ds(r, 1)], sems[r])
        for r in range(_RPW)]
    for c in in_copies:
        c.start()
    out_copies = []

    zero = jnp.zeros((_L,), jnp.float32)

    for r in range(_RPW):
        in_copies[r].wait()
        row = buf.at[r]

        def max_body(j, accs):
            base = j * (_NACC * _L)
            return tuple(
                jnp.maximum(accs[t], row[pl.ds(base + t * _L, _L)])
                for t in range(_NACC))

        accs = lax.fori_loop(
            1, _NV // _NACC, max_body,
            tuple(row[pl.ds(t * _L, _L)] for t in range(_NACC)), unroll=4)
        m = _butterfly(jnp.maximum(jnp.maximum(accs[0], accs[1]),
                                   jnp.maximum(accs[2], accs[3])),
                       jnp.maximum)
        thr = m - 1.0
        thr_s = thr[0]

        izero = jnp.zeros((_L,), jnp.int32)

        _CG = 16

        def comp_body(j, carry):
            offa, offb = carry
            base = j * (_CG * _L)
            va = [row[pl.ds(base + t * _L, _L)] for t in range(_CG)]
            vb = [row[pl.ds(_NH * _L + base + t * _L, _L)]
                  for t in range(_CG)]
            code_a, code_b = izero, izero
            for t in range(_CG):
                code_a = code_a | jnp.where(va[t] > thr, 1 << t, 0)
                code_b = code_b | jnp.where(vb[t] > thr, 1 << t, 0)
            ca = _butterfly(code_a, jnp.bitwise_or)[0] << 4
            cb = _butterfly(code_b, jnp.bitwise_or)[0] << 4
            for t in range(_CG):
                compa[pl.ds(pl.multiple_of(offa, _L), _L)] = va[t]
                compb[pl.ds(pl.multiple_of(offb, _L), _L)] = vb[t]
                offa = offa + ((ca >> t) & _L)
                offb = offb + ((cb >> t) & _L)
            return offa, offb

        offa, offb = lax.fori_loop(0, _NH // _CG, comp_body, (0, 0),
                                   unroll=2)

        sent = jnp.full((_L,), -3e38, jnp.float32)
        for t in range(_G):
            compa[pl.ds(pl.multiple_of(offa, _L) + t * _L, _L)] = sent
            compb[pl.ds(pl.multiple_of(offb, _L) + t * _L, _L)] = sent
        na_blocks = (offa + (_G * _L - 1)) // (_G * _L)
        nb_blocks = (offb + (_G * _L - 1)) // (_G * _L)

        def bis_body(i, carry):
            lo, hi = carry
            mid = 0.5 * (lo + hi)

            def inner_a(k, a):
                base = pl.multiple_of(k * (_G * _L), _L)
                for t in range(_G):
                    a = a + jnp.maximum(
                        compa[pl.ds(base + t * _L, _L)] - mid, 0.0)
                return a

            def inner_b(k, a):
                base = pl.multiple_of(k * (_G * _L), _L)
                for t in range(_G):
                    a = a + jnp.maximum(
                        compb[pl.ds(base + t * _L, _L)] - mid, 0.0)
                return a

            a = lax.fori_loop(0, na_blocks, inner_a, zero)
            a = lax.fori_loop(0, nb_blocks, inner_b, a)
            s = _butterfly(a, jnp.add)
            pred = s >= 1.0
            return jnp.where(pred, mid, lo), jnp.where(pred, hi, mid)

        lo, _hi = lax.fori_loop(0, _BISECT_ITERS, bis_body, (thr, m))

        def fin_a(k, carry):
            sa, ka = carry
            base = pl.multiple_of(k * (_G * _L), _L)
            for t in range(_G):
                v = compa[pl.ds(base + t * _L, _L)]
                sup = v > lo
                sa = sa + jnp.where(sup, v, 0.0)
                ka = ka + jnp.where(sup, 1.0, 0.0)
            return sa, ka

        def fin_b(k, carry):
            sa, ka = carry
            base = pl.multiple_of(k * (_G * _L), _L)
            for t in range(_G):
                v = compb[pl.ds(base + t * _L, _L)]
                sup = v > lo
                sa = sa + jnp.where(sup, v, 0.0)
                ka = ka + jnp.where(sup, 1.0, 0.0)
            return sa, ka

        sa, ka = lax.fori_loop(0, na_blocks, fin_a, (zero, zero))
        sa, ka = lax.fori_loop(0, nb_blocks, fin_b, (sa, ka))
        tau = (_butterfly(sa, jnp.add) - 1.0) / _butterfly(ka, jnp.add)

        def out_body(j, carry):
            base = j * (_NACC * _L)
            for t in range(_NACC):
                sl = pl.ds(base + t * _L, _L)
                row[sl] = jnp.maximum(row[sl] - tau, 0.0)
            return carry

        lax.fori_loop(0, _NV // _NACC, out_body, 0, unroll=4)

        oc = pltpu.make_async_copy(buf.at[pl.ds(r, 1)],
                                   out_hbm.at[pl.ds(row0 + r, 1)],
                                   sems[_RPW + r])
        oc.start()
        out_copies.append(oc)

    for oc in out_copies:
        oc.wait()


@functools.partial(
    pl.kernel,
    out_type=jax.ShapeDtypeStruct((_ROWS, _N), jnp.float32),
    mesh=plsc.VectorSubcoreMesh(core_axis_name="c", subcore_axis_name="s",
                                num_cores=_NC, num_subcores=_NS),
    scratch_types=[pltpu.VMEM((_RPW, _N), jnp.float32),
                   pltpu.VMEM((_N + 2 * _G * _L, ), jnp.float32),
                   pltpu.VMEM((_NH * _L + _G * _L,), jnp.float32),
                   [pltpu.SemaphoreType.DMA] * (2 * _RPW)],
)
def _sparsemax_sc(x_hbm, out_hbm, buf, compa, compb, sems):
    _sc_body(x_hbm, out_hbm, buf, compa, compb, sems)


@jax.jit
def kernel(input):
    return _sparsemax_sc(input)


# speedup vs baseline: 1.0811x; 1.0249x over previous
import functools

import jax
import jax.numpy as jnp
from jax import lax
from jax.experimental import pallas as pl
from jax.experimental.pallas import tpu as pltpu
from jax.experimental.pallas import tpu_sc as plsc

_ROWS, _N = 64, 8192
_L = 16
_NC, _NS = 2, 16
_NW = _NC * _NS
_RPW = _ROWS // _NW
_NV = _N // _L
_NH = _NV // 2
_BISECT_ITERS = 12
_NACC = 4
_G = 8


def _butterfly(v, op):
    iota = lax.iota(jnp.int32, _L)
    for k in (8, 4, 2, 1):
        v = op(v, v.at[iota ^ k].get(mode="promise_in_bounds"))
    return v


def _sc_body(x_hbm, out_hbm, buf, compa, compb, sems):
    cid = lax.axis_index("c")
    sid = lax.axis_index("s")
    wid = sid * _NC + cid
    row0 = wid * _RPW

    in_copies = [
        pltpu.make_async_copy(x_hbm.at[pl.ds(row0 + r, 1)],
                              buf.at[pl.ds(r, 1)], sems[r])
        for r in range(_RPW)]
    for c in in_copies:
        c.start()
    out_copies = []

    zero = jnp.zeros((_L,), jnp.float32)

    for r in range(_RPW):
        in_copies[r].wait()
        row = buf.at[r]

        def max_body(j, accs):
            base = j * (_NACC * _L)
            return tuple(
                jnp.maximum(accs[t], row[pl.ds(base + t * _L, _L)])
                for t in range(_NACC))

        accs = lax.fori_loop(
            1, _NV // _NACC, max_body,
            tuple(row[pl.ds(t * _L, _L)] for t in range(_NACC)), unroll=4)
        m = _butterfly(jnp.maximum(jnp.maximum(accs[0], accs[1]),
                                   jnp.maximum(accs[2], accs[3])),
                       jnp.maximum)
        thr = m - 1.0
        thr_s = thr[0]

        izero = jnp.zeros((_L,), jnp.int32)

        _CG = 16

        def comp_body(j, carry):
            offa, offb = carry
            base = j * (_CG * _L)
            va = [row[pl.ds(base + t * _L, _L)] for t in range(_CG)]
            vb = [row[pl.ds(_NH * _L + base + t * _L, _L)]
                  for t in range(_CG)]
            code_a, code_b = izero, izero
            for t in range(_CG):
                code_a = code_a | jnp.where(va[t] > thr, 1 << t, 0)
                code_b = code_b | jnp.where(vb[t] > thr, 1 << t, 0)
            ca = _butterfly(code_a, jnp.bitwise_or)[0] << 4
            cb = _butterfly(code_b, jnp.bitwise_or)[0] << 4
            for t in range(_CG):
                compa[pl.ds(pl.multiple_of(offa, _L), _L)] = va[t]
                compb[pl.ds(pl.multiple_of(offb, _L), _L)] = vb[t]
                offa = offa + ((ca >> t) & _L)
                offb = offb + ((cb >> t) & _L)
            return offa, offb

        offa, offb = lax.fori_loop(0, _NH // _CG, comp_body, (0, 0),
                                   unroll=2)

        sent = jnp.full((_L,), -3e38, jnp.float32)
        for t in range(_G):
            compa[pl.ds(pl.multiple_of(offa, _L) + t * _L, _L)] = sent
            compb[pl.ds(pl.multiple_of(offb, _L) + t * _L, _L)] = sent
        na_blocks = (offa + (_G * _L - 1)) // (_G * _L)
        nb_blocks = (offb + (_G * _L - 1)) // (_G * _L)

        def bis_body(i, carry):
            lo, hi = carry
            mid = 0.5 * (lo + hi)

            def inner_a(k, a):
                base = pl.multiple_of(k * (_G * _L), _L)
                for t in range(_G):
                    a = a + jnp.maximum(
                        compa[pl.ds(base + t * _L, _L)] - mid, 0.0)
                return a

            def inner_b(k, a):
                base = pl.multiple_of(k * (_G * _L), _L)
                for t in range(_G):
                    a = a + jnp.maximum(
                        compb[pl.ds(base + t * _L, _L)] - mid, 0.0)
                return a

            a = lax.fori_loop(0, na_blocks, inner_a, zero)
            a = lax.fori_loop(0, nb_blocks, inner_b, a)
            s = _butterfly(a, jnp.add)
            pred = s >= 1.0
            return jnp.where(pred, mid, lo), jnp.where(pred, hi, mid)

        lo, _hi = lax.fori_loop(0, _BISECT_ITERS, bis_body, (thr, m))

        def fin_a(k, carry):
            sa, ka = carry
            base = pl.multiple_of(k * (_G * _L), _L)
            for t in range(_G):
                v = compa[pl.ds(base + t * _L, _L)]
                sup = v > lo
                sa = sa + jnp.where(sup, v, 0.0)
                ka = ka + jnp.where(sup, 1.0, 0.0)
            return sa, ka

        def fin_b(k, carry):
            sa, ka = carry
            base = pl.multiple_of(k * (_G * _L), _L)
            for t in range(_G):
                v = compb[pl.ds(base + t * _L, _L)]
                sup = v > lo
                sa = sa + jnp.where(sup, v, 0.0)
                ka = ka + jnp.where(sup, 1.0, 0.0)
            return sa, ka

        sa, ka = lax.fori_loop(0, na_blocks, fin_a, (zero, zero))
        sa, ka = lax.fori_loop(0, nb_blocks, fin_b, (sa, ka))
        tau = (_butterfly(sa, jnp.add) - 1.0) / _butterfly(ka, jnp.add)

        def out_body(j, carry):
            base = j * (_NACC * _L)
            for t in range(_NACC):
                sl = pl.ds(base + t * _L, _L)
                row[sl] = jnp.maximum(row[sl] - tau, 0.0)
            return carry

        lax.fori_loop(0, _NV // _NACC, out_body, 0, unroll=4)

        oc = pltpu.make_async_copy(buf.at[pl.ds(r, 1)],
                                   out_hbm.at[pl.ds(row0 + r, 1)],
                                   sems[_RPW + r])
        oc.start()
        out_copies.append(oc)

    for oc in out_copies:
        oc.wait()


@functools.partial(
    pl.kernel,
    out_type=jax.ShapeDtypeStruct((_ROWS, _N), jnp.float32),
    mesh=plsc.VectorSubcoreMesh(core_axis_name="c", subcore_axis_name="s",
                                num_cores=_NC, num_subcores=_NS),
    scratch_types=[pltpu.VMEM((_RPW, _N), jnp.float32),
                   pltpu.VMEM((_N + 2 * _G * _L, ), jnp.float32),
                   pltpu.VMEM((_NH * _L + _G * _L,), jnp.float32),
                   [pltpu.SemaphoreType.DMA] * (2 * _RPW)],
)
def _sparsemax_sc(x_hbm, out_hbm, buf, compa, compb, sems):
    _sc_body(x_hbm, out_hbm, buf, compa, compb, sems)


@jax.jit
def kernel(input):
    return _sparsemax_sc(input)


# speedup vs baseline: 1.0884x; 1.0068x over previous
import functools

import jax
import jax.numpy as jnp
from jax import lax
from jax.experimental import pallas as pl
from jax.experimental.pallas import tpu as pltpu
from jax.experimental.pallas import tpu_sc as plsc

_ROWS, _N = 64, 8192
_L = 16
_NC, _NS = 2, 16
_NW = _NC * _NS
_RPW = _ROWS // _NW
_NV = _N // _L
_NH = _NV // 2
_BISECT_ITERS = 12
_NACC = 4
_G = 8


def _butterfly(v, op):
    iota = lax.iota(jnp.int32, _L)
    for k in (8, 4, 2, 1):
        v = op(v, v.at[iota ^ k].get(mode="promise_in_bounds"))
    return v


def _sc_body(x_hbm, out_hbm, buf, compa, compb, sems):
    cid = lax.axis_index("c")
    sid = lax.axis_index("s")
    wid = sid * _NC + cid
    row0 = wid * _RPW

    in_copies = [
        pltpu.make_async_copy(x_hbm.at[pl.ds(row0 + r, 1)],
                              buf.at[pl.ds(r, 1)], sems[r])
        for r in range(_RPW)]
    for c in in_copies:
        c.start()
    out_copies = []

    zero = jnp.zeros((_L,), jnp.float32)

    for r in range(_RPW):
        in_copies[r].wait()
        row = buf.at[r]

        def max_body(j, accs):
            base = j * (_NACC * _L)
            return tuple(
                jnp.maximum(accs[t], row[pl.ds(base + t * _L, _L)])
                for t in range(_NACC))

        accs = lax.fori_loop(
            1, _NV // _NACC, max_body,
            tuple(row[pl.ds(t * _L, _L)] for t in range(_NACC)), unroll=2)
        m = _butterfly(jnp.maximum(jnp.maximum(accs[0], accs[1]),
                                   jnp.maximum(accs[2], accs[3])),
                       jnp.maximum)
        thr = m - 1.0
        thr_s = thr[0]

        izero = jnp.zeros((_L,), jnp.int32)

        _CG = 16

        def comp_body(j, carry):
            offa, offb = carry
            base = j * (_CG * _L)
            va = [row[pl.ds(base + t * _L, _L)] for t in range(_CG)]
            vb = [row[pl.ds(_NH * _L + base + t * _L, _L)]
                  for t in range(_CG)]
            code_a, code_b = izero, izero
            for t in range(_CG):
                code_a = code_a | jnp.where(va[t] > thr, 1 << t, 0)
                code_b = code_b | jnp.where(vb[t] > thr, 1 << t, 0)
            ca = _butterfly(code_a, jnp.bitwise_or)[0] << 4
            cb = _butterfly(code_b, jnp.bitwise_or)[0] << 4
            for t in range(_CG):
                compa[pl.ds(pl.multiple_of(offa, _L), _L)] = va[t]
                compb[pl.ds(pl.multiple_of(offb, _L), _L)] = vb[t]
                offa = offa + ((ca >> t) & _L)
                offb = offb + ((cb >> t) & _L)
            return offa, offb

        offa, offb = lax.fori_loop(0, _NH // _CG, comp_body, (0, 0))

        sent = jnp.full((_L,), -3e38, jnp.float32)
        for t in range(_G):
            compa[pl.ds(pl.multiple_of(offa, _L) + t * _L, _L)] = sent
            compb[pl.ds(pl.multiple_of(offb, _L) + t * _L, _L)] = sent
        na_blocks = (offa + (_G * _L - 1)) // (_G * _L)
        nb_blocks = (offb + (_G * _L - 1)) // (_G * _L)

        def bis_body(i, carry):
            lo, hi = carry
            mid = 0.5 * (lo + hi)

            def inner_a(k, a):
                base = pl.multiple_of(k * (_G * _L), _L)
                for t in range(_G):
                    a = a + jnp.maximum(
                        compa[pl.ds(base + t * _L, _L)] - mid, 0.0)
                return a

            def inner_b(k, a):
                base = pl.multiple_of(k * (_G * _L), _L)
                for t in range(_G):
                    a = a + jnp.maximum(
                        compb[pl.ds(base + t * _L, _L)] - mid, 0.0)
                return a

            a = lax.fori_loop(0, na_blocks, inner_a, zero)
            a = lax.fori_loop(0, nb_blocks, inner_b, a)
            s = _butterfly(a, jnp.add)
            pred = s >= 1.0
            return jnp.where(pred, mid, lo), jnp.where(pred, hi, mid)

        lo, _hi = lax.fori_loop(0, _BISECT_ITERS, bis_body, (thr, m))

        def fin_a(k, carry):
            sa, ka = carry
            base = pl.multiple_of(k * (_G * _L), _L)
            for t in range(_G):
                v = compa[pl.ds(base + t * _L, _L)]
                sup = v > lo
                sa = sa + jnp.where(sup, v, 0.0)
                ka = ka + jnp.where(sup, 1.0, 0.0)
            return sa, ka

        def fin_b(k, carry):
            sa, ka = carry
            base = pl.multiple_of(k * (_G * _L), _L)
            for t in range(_G):
                v = compb[pl.ds(base + t * _L, _L)]
                sup = v > lo
                sa = sa + jnp.where(sup, v, 0.0)
                ka = ka + jnp.where(sup, 1.0, 0.0)
            return sa, ka

        sa, ka = lax.fori_loop(0, na_blocks, fin_a, (zero, zero))
        sa, ka = lax.fori_loop(0, nb_blocks, fin_b, (sa, ka))
        tau = (_butterfly(sa, jnp.add) - 1.0) / _butterfly(ka, jnp.add)

        def out_body(j, carry):
            base = j * (_NACC * _L)
            for t in range(_NACC):
                sl = pl.ds(base + t * _L, _L)
                row[sl] = jnp.maximum(row[sl] - tau, 0.0)
            return carry

        lax.fori_loop(0, _NV // _NACC, out_body, 0, unroll=2)

        oc = pltpu.make_async_copy(buf.at[pl.ds(r, 1)],
                                   out_hbm.at[pl.ds(row0 + r, 1)],
                                   sems[_RPW + r])
        oc.start()
        out_copies.append(oc)

    for oc in out_copies:
        oc.wait()


@functools.partial(
    pl.kernel,
    out_type=jax.ShapeDtypeStruct((_ROWS, _N), jnp.float32),
    mesh=plsc.VectorSubcoreMesh(core_axis_name="c", subcore_axis_name="s",
                                num_cores=_NC, num_subcores=_NS),
    scratch_types=[pltpu.VMEM((_RPW, _N), jnp.float32),
                   pltpu.VMEM((_N + 2 * _G * _L, ), jnp.float32),
                   pltpu.VMEM((_NH * _L + _G * _L,), jnp.float32),
                   [pltpu.SemaphoreType.DMA] * (2 * _RPW)],
)
def _sparsemax_sc(x_hbm, out_hbm, buf, compa, compb, sems):
    _sc_body(x_hbm, out_hbm, buf, compa, compb, sems)


@jax.jit
def kernel(input):
    return _sparsemax_sc(input)
